# baseline (speedup 1.0000x reference)
import functools

import jax
import jax.numpy as jnp
from jax import lax
from jax.experimental import pallas as pl
from jax.experimental.pallas import tpu as pltpu
from jax.experimental.pallas import tpu_sc as plsc

_EMBED = 64
_CTX = 200
_NC = 2
_NS = 16
_NW = _NC * _NS
_CHUNK = 800
_SUB = 80
_NSUB = _CHUNK // _SUB
_LANES = 16
_EG = _EMBED // _LANES


def _emb_body(tb_hbm, e_hbm, p_hbm, out_hbm, p_v, idx_v, rows_v, gsem):
    wid = lax.axis_index("s") * _NC + lax.axis_index("c")
    n_per_w = tb_hbm.shape[0] // _NW
    steps = n_per_w // _CHUNK
    base = wid * n_per_w

    pltpu.sync_copy(p_hbm, p_v)

    def step(g, carry):
        off = base + g * _CHUNK
        pltpu.sync_copy(tb_hbm.at[pl.ds(off, _CHUNK)], idx_v)
        cps = [
            pltpu.async_copy(
                e_hbm.at[idx_v.at[pl.ds(s * _SUB, _SUB)]],
                rows_v.at[pl.ds(s * _SUB, _SUB)],
                gsem,
            )
            for s in range(_NSUB)
        ]
        for cp in cps:
            cp.wait()

        def add_row(i, c):
            for j in range(_EG):
                pv = p_v[i, pl.ds(j * _LANES, _LANES)]
                for k in range(_CHUNK // _CTX):
                    r = i + k * _CTX
                    rows_v[r, pl.ds(j * _LANES, _LANES)] = (
                        rows_v[r, pl.ds(j * _LANES, _LANES)] + pv
                    )
            return c

        lax.fori_loop(0, _CTX, add_row, 0)
        pltpu.sync_copy(rows_v, out_hbm.at[pl.ds(off, _CHUNK)])
        return carry

    lax.fori_loop(0, steps, step, 0)


def kernel(token_batch, E, P):
    batch, seq = token_batch.shape
    n = batch * seq
    tb = token_batch.reshape(n).astype(jnp.int32)

    emb = pl.kernel(
        _emb_body,
        out_type=jax.ShapeDtypeStruct((n, _EMBED), jnp.float32),
        mesh=plsc.VectorSubcoreMesh(core_axis_name="c", subcore_axis_name="s"),
        scratch_types=[
            pltpu.VMEM((_CTX, _EMBED), jnp.float32),
            pltpu.VMEM((_CHUNK,), jnp.int32),
            pltpu.VMEM((_CHUNK, _EMBED), jnp.float32),
            pltpu.SemaphoreType.DMA,
        ],
        compiler_params=pltpu.CompilerParams(use_tc_tiling_on_sc=False),
    )
    out = emb(tb, E, P)
    return out.reshape(batch, seq, _EMBED)


# speedup vs baseline: 1.0761x; 1.0761x over previous
import jax
import jax.numpy as jnp
from jax import lax
from jax.experimental import pallas as pl
from jax.experimental.pallas import tpu as pltpu
from jax.experimental.pallas import tpu_sc as plsc

_EMBED = 64
_CTX = 200
_NC = 2
_NS = 16
_NW = _NC * _NS
_CHUNK = 400
_SUB = 80
_NSUB = _CHUNK // _SUB
_NBUF = 4
_LANES = 16
_EG = _EMBED // _LANES
_K = _CHUNK // _CTX


def _emb_body(tb_hbm, e_hbm, p_hbm, out_hbm, p_v, *scratch):
    idx_v = scratch[0:_NBUF]
    rows_v = scratch[_NBUF:2 * _NBUF]
    isem = scratch[2 * _NBUF:3 * _NBUF]
    gsem = scratch[3 * _NBUF:4 * _NBUF]
    osem = scratch[4 * _NBUF:5 * _NBUF]

    wid = lax.axis_index("s") * _NC + lax.axis_index("c")
    n_per_w = tb_hbm.shape[0] // _NW
    steps = n_per_w // _CHUNK
    base = wid * n_per_w

    pltpu.sync_copy(p_hbm, p_v)

    def issue_idx(g, b):
        pltpu.async_copy(tb_hbm.at[pl.ds(base + g * _CHUNK, _CHUNK)],
                         idx_v[b], isem[b])

    def wait_idx(b):
        pltpu.make_async_copy(tb_hbm.at[pl.ds(0, _CHUNK)],
                              idx_v[b], isem[b]).wait()

    def fire_gather(b):
        for s in range(_NSUB):
            pltpu.async_copy(
                e_hbm.at[idx_v[b].at[pl.ds(s * _SUB, _SUB)]],
                rows_v[b].at[pl.ds(s * _SUB, _SUB)],
                gsem[b],
            )

    def drain_gather(b):
        pltpu.make_async_copy(e_hbm.at[pl.ds(0, _CHUNK)],
                              rows_v[b], gsem[b]).wait()

    def issue_store(g, b):
        pltpu.async_copy(rows_v[b],
                         out_hbm.at[pl.ds(base + g * _CHUNK, _CHUNK)],
                         osem[b])

    def wait_store(b):
        pltpu.make_async_copy(rows_v[b],
                              out_hbm.at[pl.ds(0, _CHUNK)], osem[b]).wait()

    for b in range(_NBUF):
        issue_idx(b, b)
    wait_idx(0)
    fire_gather(0)

    def outer(i, carry):
        g0 = i * _NBUF
        for b in range(_NBUF):
            g = g0 + b
            b1 = (b + 1) % _NBUF
            drain_gather(b)
            pl.when(g + _NBUF < steps)(lambda: issue_idx(g + _NBUF, b))
            pl.when(g >= _NBUF - 1)(lambda: wait_store(b1))

            def _fire():
                wait_idx(b1)
                fire_gather(b1)
            pl.when(g + 1 < steps)(_fire)

            def add_row(p, c):
                for j in range(_EG):
                    pv = p_v[p, pl.ds(j * _LANES, _LANES)]
                    for k in range(_K):
                        r = p + k * _CTX
                        rows_v[b][r, pl.ds(j * _LANES, _LANES)] = (
                            rows_v[b][r, pl.ds(j * _LANES, _LANES)] + pv
                        )
                return c

            lax.fori_loop(0, _CTX, add_row, 0)
            issue_store(g, b)
        return carry

    lax.fori_loop(0, steps // _NBUF, outer, 0)

    for b in range(1, _NBUF):
        wait_store(b)


def kernel(token_batch, E, P):
    batch, seq = token_batch.shape
    n = batch * seq
    tb = token_batch.reshape(n).astype(jnp.int32)

    emb = pl.kernel(
        _emb_body,
        out_type=jax.ShapeDtypeStruct((n, _EMBED), jnp.float32),
        mesh=plsc.VectorSubcoreMesh(core_axis_name="c", subcore_axis_name="s"),
        scratch_types=(
            [pltpu.VMEM((_CTX, _EMBED), jnp.float32)]
            + [pltpu.VMEM((_CHUNK,), jnp.int32) for _ in range(_NBUF)]
            + [pltpu.VMEM((_CHUNK, _EMBED), jnp.float32) for _ in range(_NBUF)]
            + [pltpu.SemaphoreType.DMA for _ in range(3 * _NBUF)]
        ),
        compiler_params=pltpu.CompilerParams(use_tc_tiling_on_sc=False),
    )
    out = emb(tb, E, P)
    return out.reshape(batch, seq, _EMBED)


# speedup vs baseline: 1.0763x; 1.0001x over previous
import jax
import jax.numpy as jnp
from jax import lax
from jax.experimental import pallas as pl
from jax.experimental.pallas import tpu as pltpu
from jax.experimental.pallas import tpu_sc as plsc

_EMBED = 64
_CTX = 200
_NC = 2
_NS = 16
_NW = _NC * _NS
_CHUNK = 400
_SUB = 400
_NSUB = _CHUNK // _SUB
_NBUF = 4
_LANES = 16
_EG = _EMBED // _LANES
_K = _CHUNK // _CTX


def _emb_body(tb_hbm, e_hbm, p_hbm, out_hbm, p_v, *scratch):
    idx_v = scratch[0:_NBUF]
    rows_v = scratch[_NBUF:2 * _NBUF]
    isem = scratch[2 * _NBUF:3 * _NBUF]
    gsem = scratch[3 * _NBUF:4 * _NBUF]
    osem = scratch[4 * _NBUF:5 * _NBUF]

    wid = lax.axis_index("s") * _NC + lax.axis_index("c")
    n_per_w = tb_hbm.shape[0] // _NW
    steps = n_per_w // _CHUNK
    base = wid * n_per_w

    pltpu.sync_copy(p_hbm, p_v)

    def issue_idx(g, b):
        pltpu.async_copy(tb_hbm.at[pl.ds(base + g * _CHUNK, _CHUNK)],
                         idx_v[b], isem[b])

    def wait_idx(b):
        pltpu.make_async_copy(tb_hbm.at[pl.ds(0, _CHUNK)],
                              idx_v[b], isem[b]).wait()

    def fire_gather(b):
        for s in range(_NSUB):
            pltpu.async_copy(
                e_hbm.at[idx_v[b].at[pl.ds(s * _SUB, _SUB)]],
                rows_v[b].at[pl.ds(s * _SUB, _SUB)],
                gsem[b],
            )

    def drain_gather(b):
        pltpu.make_async_copy(e_hbm.at[pl.ds(0, _CHUNK)],
                              rows_v[b], gsem[b]).wait()

    def issue_store(g, b):
        pltpu.async_copy(rows_v[b],
                         out_hbm.at[pl.ds(base + g * _CHUNK, _CHUNK)],
                         osem[b])

    def wait_store(b):
        pltpu.make_async_copy(rows_v[b],
                              out_hbm.at[pl.ds(0, _CHUNK)], osem[b]).wait()

    for b in range(_NBUF):
        issue_idx(b, b)
    wait_idx(0)
    fire_gather(0)

    def outer(i, carry):
        g0 = i * _NBUF
        for b in range(_NBUF):
            g = g0 + b
            b1 = (b + 1) % _NBUF
            drain_gather(b)
            pl.when(g + _NBUF < steps)(lambda: issue_idx(g + _NBUF, b))
            pl.when(g >= _NBUF - 1)(lambda: wait_store(b1))

            def _fire():
                wait_idx(b1)
                fire_gather(b1)
            pl.when(g + 1 < steps)(_fire)

            def add_row(p, c):
                for j in range(_EG):
                    pv = p_v[p, pl.ds(j * _LANES, _LANES)]
                    for k in range(_K):
                        r = p + k * _CTX
                        rows_v[b][r, pl.ds(j * _LANES, _LANES)] = (
                            rows_v[b][r, pl.ds(j * _LANES, _LANES)] + pv
                        )
                return c

            lax.fori_loop(0, _CTX, add_row, 0)
            issue_store(g, b)
        return carry

    lax.fori_loop(0, steps // _NBUF, outer, 0)

    for b in range(1, _NBUF):
        wait_store(b)


def kernel(token_batch, E, P):
    batch, seq = token_batch.shape
    n = batch * seq
    tb = token_batch.reshape(n).astype(jnp.int32)

    emb = pl.kernel(
        _emb_body,
        out_type=jax.ShapeDtypeStruct((n, _EMBED), jnp.float32),
        mesh=plsc.VectorSubcoreMesh(core_axis_name="c", subcore_axis_name="s"),
        scratch_types=(
            [pltpu.VMEM((_CTX, _EMBED), jnp.float32)]
            + [pltpu.VMEM((_CHUNK,), jnp.int32) for _ in range(_NBUF)]
            + [pltpu.VMEM((_CHUNK, _EMBED), jnp.float32) for _ in range(_NBUF)]
            + [pltpu.SemaphoreType.DMA for _ in range(3 * _NBUF)]
        ),
        compiler_params=pltpu.CompilerParams(use_tc_tiling_on_sc=False),
    )
    out = emb(tb, E, P)
    return out.reshape(batch, seq, _EMBED)
